# baseline (speedup 1.0000x reference)
import functools

import jax
import jax.numpy as jnp
from jax import lax
from jax.experimental import pallas as pl
from jax.experimental.pallas import tpu as pltpu
from jax.experimental.pallas import tpu_sc as plsc

_NC = 2
_NS = 16
_L = 16
_NT = _NC * _NS
_ILV = plsc.PackFormat.INTERLEAVED


@functools.lru_cache(maxsize=None)
def _build_sc(B, N, D):
    fpt = D // _NT
    bh = B // 2
    mesh = plsc.VectorSubcoreMesh(core_axis_name="c", subcore_axis_name="s")

    def body(u_hbm, i_hbm, wt_hbm, ht_hbm, part_hbm,
             row_v, plane_v, acc_v, idx_v):
        cid = lax.axis_index("c")
        sid = lax.axis_index("s")
        fbase = (cid * _NS + sid) * fpt

        def row_dma(table, k):
            pltpu.sync_copy(table.at[k, pl.ds(0, N)], row_v)

        for j in range(fpt):
            k = fbase + j

            row_dma(wt_hbm, k)
            for h in range(2):
                pltpu.sync_copy(u_hbm.at[pl.ds(h * bh, bh)], idx_v)

                def wstep(g, carry, _h=h):
                    ua = idx_v[pl.ds(g * 2 * _L, _L)]
                    ub = idx_v[pl.ds(g * 2 * _L + _L, _L)]
                    wa = plsc.load_gather(row_v, [ua])
                    wb = plsc.load_gather(row_v, [ub])
                    plane_v[pl.ds(_h * bh + g * 2 * _L, 2 * _L)] = (
                        plsc.pack(wa, wb, format=_ILV))
                    return carry

                lax.fori_loop(0, bh // (2 * _L), wstep, 0, unroll=4)

            row_dma(ht_hbm, k)
            for h in range(2):
                pltpu.sync_copy(i_hbm.at[pl.ds(h * bh, bh)], idx_v)

                def hstep(g, carry, _h=h, _j=j):
                    pos = _h * bh + g * 2 * _L
                    ia = idx_v[pl.ds(g * 2 * _L, _L)]
                    ib = idx_v[pl.ds(g * 2 * _L + _L, _L)]
                    ha = plsc.load_gather(row_v, [ia])
                    hb = plsc.load_gather(row_v, [ib])
                    wa, wb = plsc.unpack(plane_v[pl.ds(pos, 2 * _L)],
                                         format=_ILV)
                    pa = wa * ha
                    pb = wb * hb
                    if _j > 0:
                        aa, ab = plsc.unpack(acc_v[pl.ds(pos, 2 * _L)],
                                             format=_ILV)
                        pa = pa + aa
                        pb = pb + ab
                    acc_v[pl.ds(pos, 2 * _L)] = plsc.pack(pa, pb, format=_ILV)
                    return carry

                lax.fori_loop(0, bh // (2 * _L), hstep, 0, unroll=4)

        pid = cid * _NS + sid
        pltpu.sync_copy(acc_v, part_hbm.at[pl.ds(pid * B, B)])

    return pl.kernel(
        body,
        out_type=jax.ShapeDtypeStruct((_NT * B,), jnp.bfloat16),
        mesh=mesh,
        compiler_params=pltpu.CompilerParams(
            needs_layout_passes=False, use_tc_tiling_on_sc=True),
        scratch_types=[
            pltpu.VMEM((N,), jnp.float32),
            pltpu.VMEM((B,), jnp.bfloat16),
            pltpu.VMEM((B,), jnp.bfloat16),
            pltpu.VMEM((bh,), jnp.int32),
        ],
    )


def _tc_sum(part, B):
    def body(p_ref, o_ref):
        x = p_ref[...].reshape(_NT, B).astype(jnp.float32)
        o_ref[...] = jnp.sum(x, axis=0)

    return pl.pallas_call(
        body,
        out_shape=jax.ShapeDtypeStruct((B,), jnp.float32),
    )(part)


def _perm(v):
    return v.reshape(-1, _L, 2).transpose(0, 2, 1).reshape(-1)


def kernel(x, W, H):
    xi = x.astype(jnp.int32)
    fn = _build_sc(x.shape[0], W.shape[0], W.shape[1])
    part = fn(_perm(xi[:, 0]), _perm(xi[:, 1]), W.T, H.T)
    return _tc_sum(part, x.shape[0])
